# Initial kernel scaffold; baseline (speedup 1.0000x reference)
import jax
import jax.numpy as jnp
from jax.experimental import pallas as pl


def kernel(x_input, edge_index_input, pos_edge_index_input, W_in, b_in, W_l, b_l, W_r, W_bil, b_bil):
    raise NotImplementedError("write your pallas kernel here")


# speedup vs baseline: 3.6814x; 3.6814x over previous
import functools

import jax
import jax.numpy as jnp
from jax import lax
from jax.experimental import pallas as pl
from jax.experimental.pallas import tpu as pltpu
from jax.experimental.pallas import tpu_sc as plsc

N = 10000
D = 128
E = 320000
NC = 2
NS = 16
NW = NC * NS
EPW = E // NW
CHUNK = 80
NCHUNK = EPW // CHUNK
NPAD = 10240
RPT = NPAD // NS
CNTW = 16
DC = D + CNTW
L = 16


def _tc1_body(x_in_ref, w_in_ref, b_in_ref, w_l_ref, w_r_ref, y_ref, z_ref):
    x = jnp.dot(x_in_ref[...], w_in_ref[...],
                preferred_element_type=jnp.float32) + b_in_ref[...]
    y_ref[...] = jnp.dot(x, w_l_ref[...], preferred_element_type=jnp.float32)
    z_ref[...] = jnp.dot(x, w_r_ref[...], preferred_element_type=jnp.float32)


def _tc1(x_in, w_in, b_in, w_l, w_r):
    return pl.pallas_call(
        _tc1_body,
        out_shape=[
            jax.ShapeDtypeStruct((N, D), jnp.float32),
            jax.ShapeDtypeStruct((N, D), jnp.float32),
        ],
    )(x_in, w_in, b_in, w_l, w_r)


_MESH = plsc.VectorSubcoreMesh(core_axis_name="c", subcore_axis_name="s")


@functools.partial(
    pl.kernel,
    mesh=_MESH,
    out_type=[
        jax.ShapeDtypeStruct((NC, NPAD, D), jnp.float32),
        jax.ShapeDtypeStruct((NC, NPAD, D), jnp.float32),
    ],
    scratch_types=[
        pltpu.VMEM((CHUNK,), jnp.int32),
        pltpu.VMEM((CHUNK,), jnp.int32),
        pltpu.VMEM((CHUNK, D), jnp.float32),
        pltpu.VMEM((CHUNK, D), jnp.float32),
        pltpu.VMEM_SHARED((NPAD, D), jnp.float32),
        pltpu.SemaphoreType.DMA,
    ],
)
def _sc_segment_sum(y_hbm, src_hbm, dst_hbm, zagg_hbm, ones_hbm,
                    agg_out, cnt_out,
                    src_v, dst_v, rows_v, ones_v, agg_sp, sem):
    c = lax.axis_index("c")
    s = lax.axis_index("s")
    row0 = s * RPT
    pltpu.sync_copy(zagg_hbm.at[pl.ds(row0, RPT)], agg_sp.at[pl.ds(row0, RPT)])
    pltpu.sync_copy(ones_hbm, ones_v)
    plsc.subcore_barrier()

    ebase = (c * NS + s) * EPW

    def chunk_body(i, carry):
        off = ebase + i * CHUNK
        pltpu.sync_copy(src_hbm.at[pl.ds(off, CHUNK)], src_v)
        pltpu.sync_copy(dst_hbm.at[pl.ds(off, CHUNK)], dst_v)
        pltpu.async_copy(y_hbm.at[src_v], rows_v, sem).wait()
        pltpu.sync_copy(rows_v, agg_sp.at[dst_v], add=True)
        return carry

    lax.fori_loop(0, NCHUNK, chunk_body, 0)
    plsc.subcore_barrier()
    pltpu.sync_copy(agg_sp.at[pl.ds(row0, RPT)],
                    agg_out.at[c, pl.ds(row0, RPT)])
    plsc.subcore_barrier()

    pltpu.sync_copy(zagg_hbm.at[pl.ds(row0, RPT)], agg_sp.at[pl.ds(row0, RPT)])
    plsc.subcore_barrier()

    def cnt_body(i, carry):
        off = ebase + i * CHUNK
        pltpu.sync_copy(dst_hbm.at[pl.ds(off, CHUNK)], dst_v)
        pltpu.sync_copy(ones_v, agg_sp.at[dst_v], add=True)
        return carry

    lax.fori_loop(0, NCHUNK, cnt_body, 0)
    plsc.subcore_barrier()
    pltpu.sync_copy(agg_sp.at[pl.ds(row0, RPT)],
                    cnt_out.at[c, pl.ds(row0, RPT)])


def _tc2_body(aggp_ref, cntp_ref, z_ref, b_l_ref, w_bil_ref, h_ref, h2_ref):
    agg = aggp_ref[0, :N] + aggp_ref[1, :N]
    cnt = cntp_ref[0, :N, 0:1] + cntp_ref[1, :N, 0:1]
    h = agg / jnp.maximum(cnt, 1.0) + b_l_ref[...] + z_ref[...]
    h_ref[...] = h
    h2_ref[...] = jnp.dot(h, w_bil_ref[...],
                          preferred_element_type=jnp.float32)


def _tc2(aggp, cntp, z, b_l, w_bil):
    return pl.pallas_call(
        _tc2_body,
        out_shape=[
            jax.ShapeDtypeStruct((N, D), jnp.float32),
            jax.ShapeDtypeStruct((N, D), jnp.float32),
        ],
    )(aggp, cntp, z, b_l, w_bil)


@functools.partial(
    pl.kernel,
    mesh=_MESH,
    out_type=jax.ShapeDtypeStruct((E * L,), jnp.float32),
    scratch_types=[
        pltpu.VMEM((CHUNK,), jnp.int32),
        pltpu.VMEM((CHUNK,), jnp.int32),
        pltpu.VMEM((CHUNK, D), jnp.float32),
        pltpu.VMEM((CHUNK, D), jnp.float32),
        pltpu.VMEM((CHUNK * L,), jnp.float32),
        pltpu.SemaphoreType.DMA,
        pltpu.SemaphoreType.DMA,
    ],
)
def _sc_edge_dots(h_hbm, h2_hbm, e0_hbm, e1_hbm, out_hbm,
                  i0_v, i1_v, r1_v, r2_v, acc_v, sem0, sem1):
    c = lax.axis_index("c")
    s = lax.axis_index("s")
    ebase = (c * NS + s) * EPW

    def chunk_body(i, carry):
        off = ebase + i * CHUNK
        pltpu.sync_copy(e0_hbm.at[pl.ds(off, CHUNK)], i0_v)
        pltpu.sync_copy(e1_hbm.at[pl.ds(off, CHUNK)], i1_v)
        cp0 = pltpu.async_copy(h_hbm.at[i0_v], r1_v, sem0)
        cp1 = pltpu.async_copy(h2_hbm.at[i1_v], r2_v, sem1)
        cp0.wait()
        cp1.wait()

        def edge_body(e, carry2):
            acc = r1_v[e, pl.ds(0, L)] * r2_v[e, pl.ds(0, L)]
            for k in range(1, D // L):
                acc = acc + (r1_v[e, pl.ds(k * L, L)]
                             * r2_v[e, pl.ds(k * L, L)])
            acc_v[pl.ds(e * L, L)] = acc
            return carry2

        lax.fori_loop(0, CHUNK, edge_body, 0)
        pltpu.sync_copy(acc_v, out_hbm.at[pl.ds(off * L, CHUNK * L)])
        return carry

    lax.fori_loop(0, NCHUNK, chunk_body, 0)


def _tc3_body(p_ref, b_ref, o_ref):
    i = lax.broadcasted_iota(jnp.int32, (D, 8), 0)
    j = lax.broadcasted_iota(jnp.int32, (D, 8), 1)
    sel = (i // L == j).astype(jnp.float32)
    o_ref[...] = jnp.dot(p_ref[...], sel,
                         preferred_element_type=jnp.float32) + b_ref[...]


def _tc3(psum, b_bil):
    return pl.pallas_call(
        _tc3_body,
        out_shape=jax.ShapeDtypeStruct((E // 8, 8), jnp.float32),
    )(psum, b_bil)


def kernel(x_input, edge_index_input, pos_edge_index_input,
           W_in, b_in, W_l, b_l, W_r, W_bil, b_bil):
    e0 = edge_index_input[0].astype(jnp.int32)
    e1 = edge_index_input[1].astype(jnp.int32)
    src = pos_edge_index_input[0].astype(jnp.int32)
    dst = pos_edge_index_input[1].astype(jnp.int32)

    y, z = _tc1(x_input, W_in.T, b_in.reshape(1, D), W_l.T, W_r.T)

    aggp, cntp = _sc_segment_sum(y, src, dst,
                                 jnp.zeros((NPAD, D), jnp.float32),
                                 jnp.ones((CHUNK, D), jnp.float32))

    h, h2 = _tc2(aggp, cntp, z, b_l.reshape(1, D), W_bil.reshape(D, D).T)

    psum = _sc_edge_dots(h, h2, e0, e1)
    out8 = _tc3(psum.reshape(E // 8, D), b_bil.reshape(1, 1))
    return out8.reshape(E)


# speedup vs baseline: 6.1050x; 1.6584x over previous
import functools

import jax
import jax.numpy as jnp
from jax import lax
from jax.experimental import pallas as pl
from jax.experimental.pallas import tpu as pltpu
from jax.experimental.pallas import tpu_sc as plsc

N = 10000
D = 128
E = 320000
NC = 2
NS = 16
NW = NC * NS
EPW = E // NW
CHUNK = 80
NCHUNK = EPW // CHUNK
NPAD = 10240
RPT = NPAD // NS
CNTW = 16
DC = D + CNTW
L = 16


def _tc1_body(x_in_ref, w_in_ref, b_in_ref, w_l_ref, w_r_ref, y_ref, z_ref):
    x = jnp.dot(x_in_ref[...], w_in_ref[...],
                preferred_element_type=jnp.float32) + b_in_ref[...]
    y_ref[...] = jnp.dot(x, w_l_ref[...], preferred_element_type=jnp.float32)
    z_ref[...] = jnp.dot(x, w_r_ref[...], preferred_element_type=jnp.float32)


def _tc1(x_in, w_in, b_in, w_l, w_r):
    return pl.pallas_call(
        _tc1_body,
        out_shape=[
            jax.ShapeDtypeStruct((N, D), jnp.float32),
            jax.ShapeDtypeStruct((N, D), jnp.float32),
        ],
    )(x_in, w_in, b_in, w_l, w_r)


_MESH = plsc.VectorSubcoreMesh(core_axis_name="c", subcore_axis_name="s")


CA = 40
CPP = 50
NPASS = EPW // (CPP * CA)
NBA = 2
NBB = 5


@functools.partial(
    pl.kernel,
    mesh=_MESH,
    out_type=[
        jax.ShapeDtypeStruct((NC, NPAD, D), jnp.float32),
        jax.ShapeDtypeStruct((NC, NPAD, D), jnp.float32),
    ],
    scratch_types=[
        pltpu.VMEM((CPP, CA), jnp.int32),
        pltpu.VMEM((CPP, CA), jnp.int32),
        [pltpu.VMEM((CA, D), jnp.float32)] * NBA,
        pltpu.VMEM_SHARED((NPAD, D), jnp.float32),
        [pltpu.SemaphoreType.DMA] * NBA,
        [pltpu.SemaphoreType.DMA] * NBA,
    ],
)
def _sc_segment_sum(y_hbm, src4_hbm, dst4_hbm, zagg_hbm, ones_hbm,
                    agg_out, cnt_out,
                    src_all, dst_all, rows, agg_sp, gsem, ssem):
    c = lax.axis_index("c")
    s = lax.axis_index("s")
    w = c * NS + s
    row0 = s * RPT
    pltpu.sync_copy(zagg_hbm.at[pl.ds(row0, RPT)], agg_sp.at[pl.ds(row0, RPT)])
    plsc.subcore_barrier()

    def pass_a(p, carry):
        pltpu.sync_copy(src4_hbm.at[w, p], src_all)
        pltpu.sync_copy(dst4_hbm.at[w, p], dst_all)
        for b in range(NBA):
            pltpu.async_copy(y_hbm.at[src_all.at[b]], rows[b], gsem[b])

        def grp_a(g, carry2):
            for b in range(NBA):
                j = g * NBA + b
                prev = (b - 1) % NBA
                pltpu.make_async_copy(y_hbm.at[src_all.at[j]], rows[b],
                                      gsem[b]).wait()
                pltpu.async_copy(rows[b], agg_sp.at[dst_all.at[j]], ssem[b],
                                 add=True)

                @pl.when(j > 0)
                def _():
                    pltpu.make_async_copy(rows[prev],
                                          agg_sp.at[dst_all.at[j - 1]],
                                          ssem[prev]).wait()

                @pl.when((j > 0) & (j + NBA - 1 < CPP))
                def _():
                    pltpu.async_copy(y_hbm.at[src_all.at[j + NBA - 1]],
                                     rows[prev], gsem[prev])
            return carry2

        lax.fori_loop(0, CPP // NBA, grp_a, 0)
        pltpu.make_async_copy(rows[NBA - 1],
                              agg_sp.at[dst_all.at[CPP - 1]],
                              ssem[NBA - 1]).wait()
        return carry

    lax.fori_loop(0, NPASS, pass_a, 0)
    plsc.subcore_barrier()
    pltpu.sync_copy(agg_sp.at[pl.ds(row0, RPT)],
                    agg_out.at[c, pl.ds(row0, RPT)])
    plsc.subcore_barrier()

    pltpu.sync_copy(zagg_hbm.at[pl.ds(row0, RPT)], agg_sp.at[pl.ds(row0, RPT)])
    pltpu.sync_copy(ones_hbm, rows[0])
    plsc.subcore_barrier()

    def pass_b(p, carry):
        pltpu.sync_copy(dst4_hbm.at[w, p], dst_all)

        def grp_b(g, carry2):
            for b in range(NBA):
                j = g * NBA + b

                @pl.when(g > 0)
                def _():
                    pltpu.make_async_copy(rows[0],
                                          agg_sp.at[dst_all.at[j - NBA]],
                                          ssem[b]).wait()

                pltpu.async_copy(rows[0], agg_sp.at[dst_all.at[j]], ssem[b],
                                 add=True)
            return carry2

        lax.fori_loop(0, CPP // NBA, grp_b, 0)
        for b in range(NBA):
            pltpu.make_async_copy(
                rows[0], agg_sp.at[dst_all.at[CPP - NBA + b]],
                ssem[b]).wait()
        return carry

    lax.fori_loop(0, NPASS, pass_b, 0)
    plsc.subcore_barrier()
    pltpu.sync_copy(agg_sp.at[pl.ds(row0, RPT)],
                    cnt_out.at[c, pl.ds(row0, RPT)])


def _tc2_body(aggp_ref, cntp_ref, z_ref, b_l_ref, w_bil_ref, h_ref, h2_ref):
    agg = aggp_ref[0, :N] + aggp_ref[1, :N]
    cnt = cntp_ref[0, :N, 0:1] + cntp_ref[1, :N, 0:1]
    h = agg / jnp.maximum(cnt, 1.0) + b_l_ref[...] + z_ref[...]
    h_ref[...] = h
    h2_ref[...] = jnp.dot(h, w_bil_ref[...],
                          preferred_element_type=jnp.float32)


def _tc2(aggp, cntp, z, b_l, w_bil):
    return pl.pallas_call(
        _tc2_body,
        out_shape=[
            jax.ShapeDtypeStruct((N, D), jnp.float32),
            jax.ShapeDtypeStruct((N, D), jnp.float32),
        ],
    )(aggp, cntp, z, b_l, w_bil)


@functools.partial(
    pl.kernel,
    mesh=_MESH,
    out_type=jax.ShapeDtypeStruct((E * L,), jnp.float32),
    scratch_types=[
        pltpu.VMEM((CPP, CA), jnp.int32),
        pltpu.VMEM((CPP, CA), jnp.int32),
        [pltpu.VMEM((CA, D), jnp.float32)] * NBB,
        [pltpu.VMEM((CA, D), jnp.float32)] * NBB,
        [pltpu.VMEM((CA * L,), jnp.float32)] * 2,
        [pltpu.SemaphoreType.DMA] * NBB,
        [pltpu.SemaphoreType.DMA] * NBB,
        [pltpu.SemaphoreType.DMA] * 2,
    ],
)
def _sc_edge_dots(h_hbm, h2_hbm, e04_hbm, e14_hbm, out_hbm,
                  e0_all, e1_all, r1, r2, acc, g1sem, g2sem, osem):
    c = lax.axis_index("c")
    s = lax.axis_index("s")
    w = c * NS + s
    ebase = w * EPW

    def pass_fn(p, carry):
        pltpu.sync_copy(e04_hbm.at[w, p], e0_all)
        pltpu.sync_copy(e14_hbm.at[w, p], e1_all)
        for b in range(NBB):
            pltpu.async_copy(h_hbm.at[e0_all.at[b]], r1[b], g1sem[b])
            pltpu.async_copy(h2_hbm.at[e1_all.at[b]], r2[b], g2sem[b])

        def grp(g, carry2):
            for b in range(NBB):
                j = g * NBB + b
                par = b & 1
                gchunk = p * CPP + j
                off = ebase + gchunk * CA
                pltpu.make_async_copy(h_hbm.at[e0_all.at[j]], r1[b],
                                      g1sem[b]).wait()
                pltpu.make_async_copy(h2_hbm.at[e1_all.at[j]], r2[b],
                                      g2sem[b]).wait()

                @pl.when(gchunk >= 2)
                def _():
                    pltpu.make_async_copy(
                        acc[par], out_hbm.at[pl.ds(off * L, CA * L)],
                        osem[par]).wait()

                def edge_body(e, carry3):
                    a = r1[b][e, pl.ds(0, L)] * r2[b][e, pl.ds(0, L)]
                    for k in range(1, D // L):
                        a = a + (r1[b][e, pl.ds(k * L, L)]
                                 * r2[b][e, pl.ds(k * L, L)])
                    acc[par][pl.ds(e * L, L)] = a
                    return carry3

                lax.fori_loop(0, CA, edge_body, 0)
                pltpu.async_copy(acc[par],
                                 out_hbm.at[pl.ds(off * L, CA * L)],
                                 osem[par])

                @pl.when(j + NBB < CPP)
                def _():
                    pltpu.async_copy(h_hbm.at[e0_all.at[j + NBB]], r1[b],
                                     g1sem[b])
                    pltpu.async_copy(h2_hbm.at[e1_all.at[j + NBB]], r2[b],
                                     g2sem[b])
            return carry2

        lax.fori_loop(0, CPP // NBB, grp, 0)
        return carry

    lax.fori_loop(0, NPASS, pass_fn, 0)
    for par in range(2):
        pltpu.make_async_copy(acc[par],
                              out_hbm.at[pl.ds(ebase * L, CA * L)],
                              osem[par]).wait()


def _tc3_body(p_ref, b_ref, o_ref):
    i = lax.broadcasted_iota(jnp.int32, (D, 8), 0)
    j = lax.broadcasted_iota(jnp.int32, (D, 8), 1)
    sel = (i // L == j).astype(jnp.float32)
    o_ref[...] = jnp.dot(p_ref[...], sel,
                         preferred_element_type=jnp.float32) + b_ref[...]


def _tc3(psum, b_bil):
    return pl.pallas_call(
        _tc3_body,
        out_shape=jax.ShapeDtypeStruct((E // 8, 8), jnp.float32),
    )(psum, b_bil)


def kernel(x_input, edge_index_input, pos_edge_index_input,
           W_in, b_in, W_l, b_l, W_r, W_bil, b_bil):
    e0 = edge_index_input[0].astype(jnp.int32)
    e1 = edge_index_input[1].astype(jnp.int32)
    src = pos_edge_index_input[0].astype(jnp.int32)
    dst = pos_edge_index_input[1].astype(jnp.int32)

    y, z = _tc1(x_input, W_in.T, b_in.reshape(1, D), W_l.T, W_r.T)

    aggp, cntp = _sc_segment_sum(y,
                                 src.reshape(NW, NPASS, CPP, CA),
                                 dst.reshape(NW, NPASS, CPP, CA),
                                 jnp.zeros((NPAD, D), jnp.float32),
                                 jnp.ones((CA, D), jnp.float32))

    h, h2 = _tc2(aggp, cntp, z, b_l.reshape(1, D), W_bil.reshape(D, D).T)

    psum = _sc_edge_dots(h, h2,
                         e0.reshape(NW, NPASS, CPP, CA),
                         e1.reshape(NW, NPASS, CPP, CA))
    out8 = _tc3(psum.reshape(E // 8, D), b_bil.reshape(1, 1))
    return out8.reshape(E)


# speedup vs baseline: 6.1159x; 1.0018x over previous
import functools

import jax
import jax.numpy as jnp
from jax import lax
from jax.experimental import pallas as pl
from jax.experimental.pallas import tpu as pltpu
from jax.experimental.pallas import tpu_sc as plsc

N = 10000
D = 128
E = 320000
NC = 2
NS = 16
NW = NC * NS
EPW = E // NW
CHUNK = 80
NCHUNK = EPW // CHUNK
NPAD = 10240
RPT = NPAD // NS
CNTW = 16
DC = D + CNTW
L = 16


def _tc1_body(x_in_ref, w_in_ref, b_in_ref, w_l_ref, w_r_ref, y_ref, z_ref):
    x = jnp.dot(x_in_ref[...], w_in_ref[...],
                preferred_element_type=jnp.float32) + b_in_ref[...]
    y_ref[...] = jnp.dot(x, w_l_ref[...], preferred_element_type=jnp.float32)
    z_ref[...] = jnp.dot(x, w_r_ref[...], preferred_element_type=jnp.float32)


def _tc1(x_in, w_in, b_in, w_l, w_r):
    return pl.pallas_call(
        _tc1_body,
        out_shape=[
            jax.ShapeDtypeStruct((N, D), jnp.float32),
            jax.ShapeDtypeStruct((N, D), jnp.float32),
        ],
    )(x_in, w_in, b_in, w_l, w_r)


_MESH = plsc.VectorSubcoreMesh(core_axis_name="c", subcore_axis_name="s")


CA = 40
CPP = 50
NPASS = EPW // (CPP * CA)
NBA = 2
NBB = 5


@functools.partial(
    pl.kernel,
    mesh=_MESH,
    out_type=[
        jax.ShapeDtypeStruct((NC, NPAD, D), jnp.float32),
        jax.ShapeDtypeStruct((NC, NPAD, D), jnp.float32),
    ],
    scratch_types=[
        pltpu.VMEM((CPP, CA), jnp.int32),
        pltpu.VMEM((CPP, CA), jnp.int32),
        [pltpu.VMEM((CA, D), jnp.float32)] * NBA,
        pltpu.VMEM_SHARED((NPAD, D), jnp.float32),
        [pltpu.SemaphoreType.DMA] * NBA,
        [pltpu.SemaphoreType.DMA] * NBA,
        [pltpu.SemaphoreType.DMA] * NBB,
    ],
)
def _sc_segment_sum(y_hbm, src4_hbm, dst4_hbm, zagg_hbm, ones_hbm,
                    agg_out, cnt_out,
                    src_all, dst_all, rows, agg_sp, gsem, ssem, bsem):
    c = lax.axis_index("c")
    s = lax.axis_index("s")
    w = c * NS + s
    row0 = s * RPT
    pltpu.sync_copy(zagg_hbm.at[pl.ds(row0, RPT)], agg_sp.at[pl.ds(row0, RPT)])
    plsc.subcore_barrier()

    def pass_a(p, carry):
        pltpu.sync_copy(src4_hbm.at[w, p], src_all)
        pltpu.sync_copy(dst4_hbm.at[w, p], dst_all)
        for b in range(NBA):
            pltpu.async_copy(y_hbm.at[src_all.at[b]], rows[b], gsem[b])

        def grp_a(g, carry2):
            for b in range(NBA):
                j = g * NBA + b
                prev = (b - 1) % NBA
                pltpu.make_async_copy(y_hbm.at[src_all.at[j]], rows[b],
                                      gsem[b]).wait()
                pltpu.async_copy(rows[b], agg_sp.at[dst_all.at[j]], ssem[b],
                                 add=True)

                @pl.when(j > 0)
                def _():
                    pltpu.make_async_copy(rows[prev],
                                          agg_sp.at[dst_all.at[j - 1]],
                                          ssem[prev]).wait()

                @pl.when((j > 0) & (j + NBA - 1 < CPP))
                def _():
                    pltpu.async_copy(y_hbm.at[src_all.at[j + NBA - 1]],
                                     rows[prev], gsem[prev])
            return carry2

        lax.fori_loop(0, CPP // NBA, grp_a, 0)
        pltpu.make_async_copy(rows[NBA - 1],
                              agg_sp.at[dst_all.at[CPP - 1]],
                              ssem[NBA - 1]).wait()
        return carry

    lax.fori_loop(0, NPASS, pass_a, 0)
    plsc.subcore_barrier()
    pltpu.sync_copy(agg_sp.at[pl.ds(row0, RPT)],
                    agg_out.at[c, pl.ds(row0, RPT)])
    plsc.subcore_barrier()

    pltpu.sync_copy(zagg_hbm.at[pl.ds(row0, RPT)], agg_sp.at[pl.ds(row0, RPT)])
    pltpu.sync_copy(ones_hbm, rows[0])
    plsc.subcore_barrier()

    def pass_b(p, carry):
        pltpu.sync_copy(dst4_hbm.at[w, p], dst_all)

        def grp_b(g, carry2):
            for b in range(NBB):
                j = g * NBB + b

                @pl.when(g > 0)
                def _():
                    pltpu.make_async_copy(rows[0],
                                          agg_sp.at[dst_all.at[j - NBB]],
                                          bsem[b]).wait()

                pltpu.async_copy(rows[0], agg_sp.at[dst_all.at[j]], bsem[b],
                                 add=True)
            return carry2

        lax.fori_loop(0, CPP // NBB, grp_b, 0)
        for b in range(NBB):
            pltpu.make_async_copy(
                rows[0], agg_sp.at[dst_all.at[CPP - NBB + b]],
                bsem[b]).wait()
        return carry

    lax.fori_loop(0, NPASS, pass_b, 0)
    plsc.subcore_barrier()
    pltpu.sync_copy(agg_sp.at[pl.ds(row0, RPT)],
                    cnt_out.at[c, pl.ds(row0, RPT)])


def _tc2_body(aggp_ref, cntp_ref, z_ref, b_l_ref, w_bil_ref, h_ref, h2_ref):
    agg = aggp_ref[0, :N] + aggp_ref[1, :N]
    cnt = cntp_ref[0, :N, 0:1] + cntp_ref[1, :N, 0:1]
    h = agg / jnp.maximum(cnt, 1.0) + b_l_ref[...] + z_ref[...]
    h_ref[...] = h
    h2_ref[...] = jnp.dot(h, w_bil_ref[...],
                          preferred_element_type=jnp.float32)


def _tc2(aggp, cntp, z, b_l, w_bil):
    return pl.pallas_call(
        _tc2_body,
        out_shape=[
            jax.ShapeDtypeStruct((N, D), jnp.float32),
            jax.ShapeDtypeStruct((N, D), jnp.float32),
        ],
    )(aggp, cntp, z, b_l, w_bil)


@functools.partial(
    pl.kernel,
    mesh=_MESH,
    out_type=jax.ShapeDtypeStruct((E * L,), jnp.float32),
    scratch_types=[
        pltpu.VMEM((CPP, CA), jnp.int32),
        pltpu.VMEM((CPP, CA), jnp.int32),
        [pltpu.VMEM((CA, D), jnp.float32)] * NBB,
        [pltpu.VMEM((CA, D), jnp.float32)] * NBB,
        [pltpu.VMEM((CA * L,), jnp.float32)] * 2,
        [pltpu.SemaphoreType.DMA] * NBB,
        [pltpu.SemaphoreType.DMA] * NBB,
        [pltpu.SemaphoreType.DMA] * 2,
    ],
)
def _sc_edge_dots(h_hbm, h2_hbm, e04_hbm, e14_hbm, out_hbm,
                  e0_all, e1_all, r1, r2, acc, g1sem, g2sem, osem):
    c = lax.axis_index("c")
    s = lax.axis_index("s")
    w = c * NS + s
    ebase = w * EPW

    def pass_fn(p, carry):
        pltpu.sync_copy(e04_hbm.at[w, p], e0_all)
        pltpu.sync_copy(e14_hbm.at[w, p], e1_all)
        for b in range(NBB):
            pltpu.async_copy(h_hbm.at[e0_all.at[b]], r1[b], g1sem[b])
            pltpu.async_copy(h2_hbm.at[e1_all.at[b]], r2[b], g2sem[b])

        def grp(g, carry2):
            for b in range(NBB):
                j = g * NBB + b
                par = b & 1
                gchunk = p * CPP + j
                off = ebase + gchunk * CA
                pltpu.make_async_copy(h_hbm.at[e0_all.at[j]], r1[b],
                                      g1sem[b]).wait()
                pltpu.make_async_copy(h2_hbm.at[e1_all.at[j]], r2[b],
                                      g2sem[b]).wait()

                @pl.when(gchunk >= 2)
                def _():
                    pltpu.make_async_copy(
                        acc[par], out_hbm.at[pl.ds(off * L, CA * L)],
                        osem[par]).wait()

                def edge_body(e, carry3):
                    a = r1[b][e, pl.ds(0, L)] * r2[b][e, pl.ds(0, L)]
                    for k in range(1, D // L):
                        a = a + (r1[b][e, pl.ds(k * L, L)]
                                 * r2[b][e, pl.ds(k * L, L)])
                    acc[par][pl.ds(e * L, L)] = a
                    return carry3

                lax.fori_loop(0, CA, edge_body, 0)
                pltpu.async_copy(acc[par],
                                 out_hbm.at[pl.ds(off * L, CA * L)],
                                 osem[par])

                @pl.when(j + NBB < CPP)
                def _():
                    pltpu.async_copy(h_hbm.at[e0_all.at[j + NBB]], r1[b],
                                     g1sem[b])
                    pltpu.async_copy(h2_hbm.at[e1_all.at[j + NBB]], r2[b],
                                     g2sem[b])
            return carry2

        lax.fori_loop(0, CPP // NBB, grp, 0)
        return carry

    lax.fori_loop(0, NPASS, pass_fn, 0)
    for par in range(2):
        pltpu.make_async_copy(acc[par],
                              out_hbm.at[pl.ds(ebase * L, CA * L)],
                              osem[par]).wait()


def _tc3_body(p_ref, b_ref, o_ref):
    i = lax.broadcasted_iota(jnp.int32, (D, 8), 0)
    j = lax.broadcasted_iota(jnp.int32, (D, 8), 1)
    sel = (i // L == j).astype(jnp.float32)
    o_ref[...] = jnp.dot(p_ref[...], sel,
                         preferred_element_type=jnp.float32) + b_ref[...]


def _tc3(psum, b_bil):
    return pl.pallas_call(
        _tc3_body,
        out_shape=jax.ShapeDtypeStruct((E // 8, 8), jnp.float32),
    )(psum, b_bil)


def kernel(x_input, edge_index_input, pos_edge_index_input,
           W_in, b_in, W_l, b_l, W_r, W_bil, b_bil):
    e0 = edge_index_input[0].astype(jnp.int32)
    e1 = edge_index_input[1].astype(jnp.int32)
    src = pos_edge_index_input[0].astype(jnp.int32)
    dst = pos_edge_index_input[1].astype(jnp.int32)

    y, z = _tc1(x_input, W_in.T, b_in.reshape(1, D), W_l.T, W_r.T)

    aggp, cntp = _sc_segment_sum(y,
                                 src.reshape(NW, NPASS, CPP, CA),
                                 dst.reshape(NW, NPASS, CPP, CA),
                                 jnp.zeros((NPAD, D), jnp.float32),
                                 jnp.ones((CA, D), jnp.float32))

    h, h2 = _tc2(aggp, cntp, z, b_l.reshape(1, D), W_bil.reshape(D, D).T)

    psum = _sc_edge_dots(h, h2,
                         e0.reshape(NW, NPASS, CPP, CA),
                         e1.reshape(NW, NPASS, CPP, CA))
    out8 = _tc3(psum.reshape(E // 8, D), b_bil.reshape(1, 1))
    return out8.reshape(E)


# speedup vs baseline: 7.0563x; 1.1538x over previous
import functools

import jax
import jax.numpy as jnp
from jax import lax
from jax.experimental import pallas as pl
from jax.experimental.pallas import tpu as pltpu
from jax.experimental.pallas import tpu_sc as plsc

N = 10000
D = 128
E = 320000
NC = 2
NS = 16
NW = NC * NS
EPW = E // NW
CHUNK = 80
NCHUNK = EPW // CHUNK
NPAD = 10240
RPT = NPAD // NS
CNTW = 16
DC = D + CNTW
L = 16


def _tc1_body(x_in_ref, w_in_ref, b_in_ref, w_l_ref, w_r_ref, y_ref, z_ref):
    x = jnp.dot(x_in_ref[...], w_in_ref[...],
                preferred_element_type=jnp.float32) + b_in_ref[...]
    y_ref[...] = jnp.dot(x, w_l_ref[...], preferred_element_type=jnp.float32)
    z_ref[...] = jnp.dot(x, w_r_ref[...], preferred_element_type=jnp.float32)


def _tc1(x_in, w_in, b_in, w_l, w_r):
    return pl.pallas_call(
        _tc1_body,
        out_shape=[
            jax.ShapeDtypeStruct((N, D), jnp.float32),
            jax.ShapeDtypeStruct((N, D), jnp.float32),
        ],
    )(x_in, w_in, b_in, w_l, w_r)


_MESH = plsc.VectorSubcoreMesh(core_axis_name="c", subcore_axis_name="s")


CA = 40
CPP = 50
CAA = 80
CPPA = 25
NPASS = 5
NBA = 2
NBB = 5


@functools.partial(
    pl.kernel,
    mesh=_MESH,
    out_type=[
        jax.ShapeDtypeStruct((NC, NPAD, D), jnp.float32),
        jax.ShapeDtypeStruct((NC, NPAD, D), jnp.float32),
    ],
    scratch_types=[
        pltpu.VMEM((CPPA, CAA), jnp.int32),
        pltpu.VMEM((CPPA, CAA), jnp.int32),
        [pltpu.VMEM((CAA, D), jnp.float32)] * NBA,
        pltpu.VMEM_SHARED((NPAD, D), jnp.float32),
        [pltpu.SemaphoreType.DMA] * NBA,
        [pltpu.SemaphoreType.DMA] * NBA,
        [pltpu.SemaphoreType.DMA] * NBB,
    ],
)
def _sc_segment_sum(y_hbm, src4_hbm, dst4_hbm, zagg_hbm, ones_hbm,
                    agg_out, cnt_out,
                    src_all, dst_all, rows, agg_sp, gsem, ssem, bsem):
    c = lax.axis_index("c")
    s = lax.axis_index("s")
    w = c * NS + s
    row0 = s * RPT
    pltpu.sync_copy(zagg_hbm.at[pl.ds(row0, RPT)], agg_sp.at[pl.ds(row0, RPT)])
    plsc.subcore_barrier()

    def pass_a(p, carry):
        pltpu.sync_copy(src4_hbm.at[w, p], src_all)
        pltpu.sync_copy(dst4_hbm.at[w, p], dst_all)
        for b in range(NBA):
            pltpu.async_copy(y_hbm.at[src_all.at[b]], rows[b], gsem[b])

        def grp_a(g, carry2):
            for b in range(NBA):
                j = g * NBA + b
                prev = (b - 1) % NBA
                pltpu.make_async_copy(y_hbm.at[src_all.at[j]], rows[b],
                                      gsem[b]).wait()
                pltpu.async_copy(rows[b], agg_sp.at[dst_all.at[j]], ssem[b],
                                 add=True)

                @pl.when(j > 0)
                def _():
                    pltpu.make_async_copy(rows[prev],
                                          agg_sp.at[dst_all.at[j - 1]],
                                          ssem[prev]).wait()

                @pl.when((j > 0) & (j + NBA - 1 < CPPA))
                def _():
                    pltpu.async_copy(y_hbm.at[src_all.at[j + NBA - 1]],
                                     rows[prev], gsem[prev])
            return carry2

        lax.fori_loop(0, CPPA // NBA, grp_a, 0)
        jt = CPPA - 1
        pltpu.make_async_copy(y_hbm.at[src_all.at[jt]], rows[0],
                              gsem[0]).wait()
        pltpu.async_copy(rows[0], agg_sp.at[dst_all.at[jt]], ssem[0],
                         add=True)
        pltpu.make_async_copy(rows[1], agg_sp.at[dst_all.at[jt - 1]],
                              ssem[1]).wait()
        pltpu.make_async_copy(rows[0], agg_sp.at[dst_all.at[jt]],
                              ssem[0]).wait()
        return carry

    lax.fori_loop(0, NPASS, pass_a, 0)
    plsc.subcore_barrier()
    pltpu.sync_copy(agg_sp.at[pl.ds(row0, RPT)],
                    agg_out.at[c, pl.ds(row0, RPT)])
    plsc.subcore_barrier()

    pltpu.sync_copy(zagg_hbm.at[pl.ds(row0, RPT)], agg_sp.at[pl.ds(row0, RPT)])
    pltpu.sync_copy(ones_hbm, rows[0])
    plsc.subcore_barrier()

    def pass_b(p, carry):
        pltpu.sync_copy(dst4_hbm.at[w, p], dst_all)

        def grp_b(g, carry2):
            for b in range(NBB):
                j = g * NBB + b

                @pl.when(g > 0)
                def _():
                    pltpu.make_async_copy(rows[0],
                                          agg_sp.at[dst_all.at[j - NBB]],
                                          bsem[b]).wait()

                pltpu.async_copy(rows[0], agg_sp.at[dst_all.at[j]], bsem[b],
                                 add=True)
            return carry2

        lax.fori_loop(0, CPPA // NBB, grp_b, 0)
        for b in range(NBB):
            pltpu.make_async_copy(
                rows[0], agg_sp.at[dst_all.at[CPPA - NBB + b]],
                bsem[b]).wait()
        return carry

    lax.fori_loop(0, NPASS, pass_b, 0)
    plsc.subcore_barrier()
    pltpu.sync_copy(agg_sp.at[pl.ds(row0, RPT)],
                    cnt_out.at[c, pl.ds(row0, RPT)])


def _tc2_body(aggp_ref, cntp_ref, z_ref, b_l_ref, w_bil_ref, h_ref, h2_ref):
    agg = aggp_ref[0, :N] + aggp_ref[1, :N]
    cnt = cntp_ref[0, :N, 0:1] + cntp_ref[1, :N, 0:1]
    h = agg / jnp.maximum(cnt, 1.0) + b_l_ref[...] + z_ref[...]
    h_ref[...] = h
    h2_ref[...] = jnp.dot(h, w_bil_ref[...],
                          preferred_element_type=jnp.float32)


def _tc2(aggp, cntp, z, b_l, w_bil):
    return pl.pallas_call(
        _tc2_body,
        out_shape=[
            jax.ShapeDtypeStruct((N, D), jnp.float32),
            jax.ShapeDtypeStruct((N, D), jnp.float32),
        ],
    )(aggp, cntp, z, b_l, w_bil)


@functools.partial(
    pl.kernel,
    mesh=_MESH,
    out_type=jax.ShapeDtypeStruct((E * L,), jnp.float32),
    scratch_types=[
        pltpu.VMEM((CPP, CA), jnp.int32),
        pltpu.VMEM((CPP, CA), jnp.int32),
        [pltpu.VMEM((CA, D), jnp.float32)] * NBB,
        [pltpu.VMEM((CA, D), jnp.float32)] * NBB,
        [pltpu.VMEM((CA * L,), jnp.float32)] * 2,
        [pltpu.SemaphoreType.DMA] * NBB,
        [pltpu.SemaphoreType.DMA] * NBB,
        [pltpu.SemaphoreType.DMA] * 2,
    ],
)
def _sc_edge_dots(h_hbm, h2_hbm, e04_hbm, e14_hbm, out_hbm,
                  e0_all, e1_all, r1, r2, acc, g1sem, g2sem, osem):
    c = lax.axis_index("c")
    s = lax.axis_index("s")
    w = c * NS + s
    ebase = w * EPW

    def pass_fn(p, carry):
        pltpu.sync_copy(e04_hbm.at[w, p], e0_all)
        pltpu.sync_copy(e14_hbm.at[w, p], e1_all)
        for b in range(NBB):
            pltpu.async_copy(h_hbm.at[e0_all.at[b]], r1[b], g1sem[b])
            pltpu.async_copy(h2_hbm.at[e1_all.at[b]], r2[b], g2sem[b])

        def grp(g, carry2):
            for b in range(NBB):
                j = g * NBB + b
                par = b & 1
                gchunk = p * CPP + j
                off = ebase + gchunk * CA
                pltpu.make_async_copy(h_hbm.at[e0_all.at[j]], r1[b],
                                      g1sem[b]).wait()
                pltpu.make_async_copy(h2_hbm.at[e1_all.at[j]], r2[b],
                                      g2sem[b]).wait()

                @pl.when(gchunk >= 2)
                def _():
                    pltpu.make_async_copy(
                        acc[par], out_hbm.at[pl.ds(off * L, CA * L)],
                        osem[par]).wait()

                def edge_body(e, carry3):
                    a = r1[b][e, pl.ds(0, L)] * r2[b][e, pl.ds(0, L)]
                    for k in range(1, D // L):
                        a = a + (r1[b][e, pl.ds(k * L, L)]
                                 * r2[b][e, pl.ds(k * L, L)])
                    acc[par][pl.ds(e * L, L)] = a
                    return carry3

                lax.fori_loop(0, CA, edge_body, 0)
                pltpu.async_copy(acc[par],
                                 out_hbm.at[pl.ds(off * L, CA * L)],
                                 osem[par])

                @pl.when(j + NBB < CPP)
                def _():
                    pltpu.async_copy(h_hbm.at[e0_all.at[j + NBB]], r1[b],
                                     g1sem[b])
                    pltpu.async_copy(h2_hbm.at[e1_all.at[j + NBB]], r2[b],
                                     g2sem[b])
            return carry2

        lax.fori_loop(0, CPP // NBB, grp, 0)
        return carry

    lax.fori_loop(0, NPASS, pass_fn, 0)
    for par in range(2):
        pltpu.make_async_copy(acc[par],
                              out_hbm.at[pl.ds(ebase * L, CA * L)],
                              osem[par]).wait()


def _tc3_body(p_ref, b_ref, o_ref):
    i = lax.broadcasted_iota(jnp.int32, (D, 8), 0)
    j = lax.broadcasted_iota(jnp.int32, (D, 8), 1)
    sel = (i // L == j).astype(jnp.float32)
    o_ref[...] = jnp.dot(p_ref[...], sel,
                         preferred_element_type=jnp.float32) + b_ref[...]


def _tc3(psum, b_bil):
    return pl.pallas_call(
        _tc3_body,
        out_shape=jax.ShapeDtypeStruct((E // 8, 8), jnp.float32),
    )(psum, b_bil)


def kernel(x_input, edge_index_input, pos_edge_index_input,
           W_in, b_in, W_l, b_l, W_r, W_bil, b_bil):
    e0 = edge_index_input[0].astype(jnp.int32)
    e1 = edge_index_input[1].astype(jnp.int32)
    src = pos_edge_index_input[0].astype(jnp.int32)
    dst = pos_edge_index_input[1].astype(jnp.int32)

    y, z = _tc1(x_input, W_in.T, b_in.reshape(1, D), W_l.T, W_r.T)

    aggp, cntp = _sc_segment_sum(y,
                                 src.reshape(NW, NPASS, CPPA, CAA),
                                 dst.reshape(NW, NPASS, CPPA, CAA),
                                 jnp.zeros((NPAD, D), jnp.float32),
                                 jnp.ones((CAA, D), jnp.float32))

    h, h2 = _tc2(aggp, cntp, z, b_l.reshape(1, D), W_bil.reshape(D, D).T)

    psum = _sc_edge_dots(h, h2,
                         e0.reshape(NW, NPASS, CPP, CA),
                         e1.reshape(NW, NPASS, CPP, CA))
    out8 = _tc3(psum.reshape(E // 8, D), b_bil.reshape(1, 1))
    return out8.reshape(E)
